# Initial kernel scaffold; baseline (speedup 1.0000x reference)
import jax
import jax.numpy as jnp
from jax.experimental import pallas as pl


def kernel(x, edge_index, edge_weight, W1, b1, W2, b2, W3, b3, W4, b4):
    raise NotImplementedError("write your pallas kernel here")


# speedup vs baseline: 5.1781x; 5.1781x over previous
import functools

import jax
import jax.numpy as jnp
from jax import lax
from jax.experimental import pallas as pl
from jax.experimental.pallas import tpu as pltpu
from jax.experimental.pallas import tpu_sc as plsc

N = 10000
E = 320000
D = 128
HD = D // 2
NPAD = 10240
NT = 16
NC = 2
ROWS_PER_TILE = NPAD // NT
CHUNK = 80
NCHUNK = ROWS_PER_TILE // CHUNK
EPW = E // (NT * NC)
KW = EPW // CHUNK
EPT = E // NT
KT = EPT // CHUNK

_mesh = plsc.VectorSubcoreMesh(core_axis_name="c", subcore_axis_name="s")
_sc_params = pltpu.CompilerParams(needs_layout_passes=False, use_tc_tiling_on_sc=False)


def _zero_buf(buf, nrow):
    z = jnp.zeros((16,), jnp.float32)

    def row(i, _):
        for j in range(buf.shape[1] // 16):
            buf[i, pl.ds(16 * j, 16)] = z
        return 0

    lax.fori_loop(0, nrow, row, 0)


@functools.partial(
    pl.kernel,
    out_type=jax.ShapeDtypeStruct((NC, NPAD, 16), jnp.float32),
    mesh=_mesh,
    compiler_params=_sc_params,
    scratch_types=[
        pltpu.VMEM((KW, CHUNK), jnp.int32),
        pltpu.VMEM((KW, CHUNK), jnp.float32),
        pltpu.VMEM((CHUNK, 16), jnp.float32),
        pltpu.VMEM_SHARED((NPAD, 16), jnp.float32),
    ],
)
def _deg_kernel(dst_hbm, ew_hbm, out_hbm, dstv, ewv, sbuf, degmat):
    c = lax.axis_index("c")
    t = lax.axis_index("s")
    wid = t * NC + c
    base = t * ROWS_PER_TILE

    _zero_buf(sbuf, CHUNK)

    def zrow(m, _):
        pltpu.sync_copy(sbuf, degmat.at[pl.ds(base + CHUNK * m, CHUNK)])
        return 0

    lax.fori_loop(0, NCHUNK, zrow, 0)
    plsc.subcore_barrier()

    pltpu.sync_copy(dst_hbm.at[wid], dstv)
    pltpu.sync_copy(ew_hbm.at[wid], ewv)

    def edge_chunk(k, _):
        def grp(g, _):
            ew16 = ewv[k, pl.ds(16 * g, 16)]
            for i in range(16):
                sbuf[16 * g + i] = jnp.full((16,), ew16[i])
            return 0

        lax.fori_loop(0, CHUNK // 16, grp, 0)
        pltpu.sync_copy(sbuf, degmat.at[dstv.at[k]], add=True)
        return 0

    lax.fori_loop(0, KW, edge_chunk, 0)
    plsc.subcore_barrier()

    def out_chunk(m, _):
        rb = base + CHUNK * m
        pltpu.sync_copy(degmat.at[pl.ds(rb, CHUNK)], out_hbm.at[c, pl.ds(rb, CHUNK)])
        return 0

    lax.fori_loop(0, NCHUNK, out_chunk, 0)


def _node_body(dp0_ref, dp1_ref, dinv_ref, dinv2_ref):
    deg = 1.0 + dp0_ref[...] + dp1_ref[...]
    dinv = jnp.where(deg > 0, lax.rsqrt(deg), 0.0)
    dinv_ref[...] = dinv
    dinv2_ref[...] = dinv * dinv


def _node_kernel(dp0, dp1):
    return pl.pallas_call(
        _node_body,
        out_shape=(
            jax.ShapeDtypeStruct((100, 100), jnp.float32),
            jax.ShapeDtypeStruct((100, 100), jnp.float32),
        ),
    )(dp0, dp1)


@functools.partial(
    pl.kernel,
    out_type=jax.ShapeDtypeStruct((NT * NC, KW, CHUNK), jnp.float32),
    mesh=_mesh,
    compiler_params=_sc_params,
    scratch_types=[
        pltpu.VMEM((N,), jnp.float32),
        pltpu.VMEM((KW, CHUNK), jnp.int32),
        pltpu.VMEM((KW, CHUNK), jnp.int32),
        pltpu.VMEM((KW, CHUNK), jnp.float32),
        pltpu.VMEM((KW, CHUNK), jnp.float32),
    ],
)
def _norm_kernel(src_hbm, dst_hbm, ew_hbm, dinv_hbm, out_hbm,
                 dinvv, srcv, dstv, ewv, outv):
    c = lax.axis_index("c")
    t = lax.axis_index("s")
    wid = t * NC + c

    pltpu.sync_copy(dinv_hbm, dinvv)
    pltpu.sync_copy(src_hbm.at[wid], srcv)
    pltpu.sync_copy(dst_hbm.at[wid], dstv)
    pltpu.sync_copy(ew_hbm.at[wid], ewv)

    def chunk(k, _):
        for g in range(CHUNK // 16):
            sl = pl.ds(16 * g, 16)
            ds_ = plsc.load_gather(dinvv, [srcv[k, sl]])
            dd = plsc.load_gather(dinvv, [dstv[k, sl]])
            outv[k, sl] = ds_ * ewv[k, sl] * dd
        return 0

    lax.fori_loop(0, KW, chunk, 0)
    pltpu.sync_copy(outv, out_hbm.at[wid])


@functools.partial(
    pl.kernel,
    out_type=jax.ShapeDtypeStruct((NC, NPAD, HD), jnp.float32),
    mesh=_mesh,
    compiler_params=_sc_params,
    scratch_types=[
        pltpu.VMEM((KT, CHUNK), jnp.int32),
        pltpu.VMEM((KT, CHUNK), jnp.int32),
        pltpu.VMEM((KT, CHUNK), jnp.float32),
        pltpu.VMEM((CHUNK, HD), jnp.float32),
        pltpu.VMEM((CHUNK, HD), jnp.float32),
        pltpu.VMEM((CHUNK,), jnp.float32),
        pltpu.VMEM((HD,), jnp.float32),
        pltpu.VMEM_SHARED((NPAD, HD), jnp.float32),
        pltpu.SemaphoreType.DMA,
    ],
)
def _scatter_kernel(h_hbm, src_hbm, dst_hbm, norm_hbm, d2_hbm, b_hbm, out_hbm,
                    srcv, dstv, normv, rows, zbuf, d2v, bv, acc, sem):
    c = lax.axis_index("c")
    t = lax.axis_index("s")
    base = t * ROWS_PER_TILE
    hoff = c * N

    _zero_buf(zbuf, CHUNK)
    pltpu.sync_copy(b_hbm.at[pl.ds(HD * c, HD)], bv)

    def init_chunk(m, _):
        rb = base + CHUNK * m

        @pl.when(rb < N)
        def _():
            pltpu.async_copy(h_hbm.at[pl.ds(hoff + rb, CHUNK)], rows, sem).wait()
            pltpu.sync_copy(d2_hbm.at[pl.ds(rb, CHUNK)], d2v)

            def grp(g, _):
                d16 = d2v[pl.ds(16 * g, 16)]
                for i in range(16):
                    r = 16 * g + i
                    dv = jnp.full((16,), d16[i])
                    for j in range(HD // 16):
                        sl = pl.ds(16 * j, 16)
                        rows[r, sl] = rows[r, sl] * dv + bv[sl]
                return 0

            lax.fori_loop(0, CHUNK // 16, grp, 0)
            pltpu.sync_copy(rows, acc.at[pl.ds(rb, CHUNK)])

        @pl.when(rb >= N)
        def _():
            pltpu.sync_copy(zbuf, acc.at[pl.ds(rb, CHUNK)])
        return 0

    lax.fori_loop(0, NCHUNK, init_chunk, 0)
    plsc.subcore_barrier()

    pltpu.sync_copy(src_hbm.at[t], srcv)
    pltpu.sync_copy(dst_hbm.at[t], dstv)
    pltpu.sync_copy(norm_hbm.at[t], normv)

    @pl.when(c > 0)
    def _():
        def shift(k, _):
            for g in range(CHUNK // 16):
                sl = pl.ds(16 * g, 16)
                srcv[k, sl] = srcv[k, sl] + N
            return 0

        lax.fori_loop(0, KT, shift, 0)

    def edge_chunk(k, _):
        pltpu.async_copy(h_hbm.at[srcv.at[k]], rows, sem).wait()

        def grp(g, _):
            n16 = normv[k, pl.ds(16 * g, 16)]
            for i in range(16):
                r = 16 * g + i
                nv = jnp.full((16,), n16[i])
                for j in range(HD // 16):
                    sl = pl.ds(16 * j, 16)
                    rows[r, sl] = rows[r, sl] * nv
            return 0

        lax.fori_loop(0, CHUNK // 16, grp, 0)
        pltpu.sync_copy(rows, acc.at[dstv.at[k]], add=True)
        return 0

    lax.fori_loop(0, KT, edge_chunk, 0)
    plsc.subcore_barrier()

    def out_chunk(m, _):
        rb = base + CHUNK * m
        pltpu.sync_copy(acc.at[pl.ds(rb, CHUNK)], out_hbm.at[c, pl.ds(rb, CHUNK)])
        return 0

    lax.fori_loop(0, NCHUNK, out_chunk, 0)


def _mm_body(x_ref, w_ref, o_ref):
    o_ref[0] = jnp.dot(x_ref[...], w_ref[0],
                       preferred_element_type=jnp.float32,
                       precision=lax.Precision.HIGHEST)


def _mm(x, w):
    ws = jnp.stack([w[:, :HD], w[:, HD:]])
    return pl.pallas_call(
        _mm_body,
        grid=(NC, 10),
        in_specs=[
            pl.BlockSpec((1000, D), lambda f, i: (i, 0)),
            pl.BlockSpec((1, D, HD), lambda f, i: (f, 0, 0)),
        ],
        out_specs=pl.BlockSpec((1, 1000, HD), lambda f, i: (f, i, 0)),
        out_shape=jax.ShapeDtypeStruct((NC, N, HD), jnp.float32),
    )(x, ws)


def _fused_body(s_ref, w_ref, o_ref):
    a = jnp.maximum(jnp.concatenate([s_ref[0], s_ref[1]], axis=1), 0.0)
    o_ref[0] = jnp.dot(a, w_ref[0],
                       preferred_element_type=jnp.float32,
                       precision=lax.Precision.HIGHEST)


def _fused(s, w):
    ws = jnp.stack([w[:, :HD], w[:, HD:]])
    return pl.pallas_call(
        _fused_body,
        grid=(NC, 10),
        in_specs=[
            pl.BlockSpec((NC, 1000, HD), lambda f, i: (0, i, 0)),
            pl.BlockSpec((1, D, HD), lambda f, i: (f, 0, 0)),
        ],
        out_specs=pl.BlockSpec((1, 1000, HD), lambda f, i: (f, i, 0)),
        out_shape=jax.ShapeDtypeStruct((NC, N, HD), jnp.float32),
    )(s, ws)


def _final_body(s_ref, o_ref):
    z = jnp.concatenate([s_ref[0], s_ref[1]], axis=1)
    m = jnp.max(z, axis=1, keepdims=True)
    ez = jnp.exp(z - m)
    o_ref[...] = (z - m) - jnp.log(jnp.sum(ez, axis=1, keepdims=True))


def _final(s):
    return pl.pallas_call(
        _final_body,
        grid=(10,),
        in_specs=[pl.BlockSpec((NC, 1000, HD), lambda i: (0, i, 0))],
        out_specs=pl.BlockSpec((1000, D), lambda i: (i, 0)),
        out_shape=jax.ShapeDtypeStruct((N, D), jnp.float32),
    )(s)


def kernel(x, edge_index, edge_weight, W1, b1, W2, b2, W3, b3, W4, b4):
    srcw = edge_index[0].reshape(NT * NC, KW, CHUNK)
    dstw = edge_index[1].reshape(NT * NC, KW, CHUNK)
    eww = edge_weight.reshape(NT * NC, KW, CHUNK)
    srct = edge_index[0].reshape(NT, KT, CHUNK)
    dstt = edge_index[1].reshape(NT, KT, CHUNK)

    degpart = _deg_kernel(dstw, eww)
    dp = degpart[:, :N, 0].reshape(NC, 100, 100)
    dinv_m, dinv2_m = _node_kernel(dp[0], dp[1])
    dinv = dinv_m.reshape(N)
    dinv2 = dinv2_m.reshape(N)
    normt = _norm_kernel(srcw, dstw, eww, dinv).reshape(NT, KT, CHUNK)

    def layer(h, b):
        h2 = h.reshape(NC * N, HD)
        return _scatter_kernel(h2, srct, dstt, normt, dinv2, b)

    h = _mm(x, W1)
    s = layer(h, b1)
    h = _fused(s, W2)
    s = layer(h, b2)
    h = _fused(s, W3)
    s = layer(h, b3)
    h = _fused(s, W4)
    s = layer(h, b4)
    return _final(s)
